# baseline (speedup 1.0000x reference)
import jax
import jax.numpy as jnp
from jax.experimental import pallas as pl

NUM_ROWS = 5
EMB = 64
BLOCK_B = 256


def _embed_block(idx_ref, tab_ref, out_ref):
    b, s = idx_ref.shape
    idx = idx_ref[...][..., None]
    rows = jax.lax.broadcasted_iota(jnp.int32, (1, 1, NUM_ROWS), 2)
    one_hot = (idx == rows).astype(jnp.float32).reshape(b * s, NUM_ROWS)
    out = jax.lax.dot_general(
        one_hot, tab_ref[...],
        dimension_numbers=(((1,), (0,)), ((), ())),
        preferred_element_type=jnp.float32,
    )
    out_ref[...] = out.reshape(b, s, EMB)


@jax.jit
def kernel(inputs, table):
    batch, seq = inputs.shape
    grid = (batch // BLOCK_B,)
    return pl.pallas_call(
        _embed_block,
        grid=grid,
        in_specs=[
            pl.BlockSpec((BLOCK_B, seq), lambda i: (i, 0)),
            pl.BlockSpec((NUM_ROWS, EMB), lambda i: (0, 0)),
        ],
        out_specs=pl.BlockSpec((BLOCK_B, seq, EMB), lambda i: (i, 0, 0)),
        out_shape=jax.ShapeDtypeStruct((batch, seq, EMB), table.dtype),
    )(inputs, table)


# speedup vs baseline: 1.0080x; 1.0080x over previous
import jax
import jax.numpy as jnp
from jax.experimental import pallas as pl
from jax.experimental.pallas import tpu as pltpu

NUM_ROWS = 5
EMB = 64
BLOCK_B = 256


def _embed_block(idx_ref, tab_ref, out_ref):
    idx = idx_ref[...][..., None]
    tab = tab_ref[...]
    acc = jnp.broadcast_to(tab[0].reshape(1, 1, EMB), out_ref.shape)
    for r in range(1, NUM_ROWS):
        acc = jnp.where(idx == r, tab[r].reshape(1, 1, EMB), acc)
    out_ref[...] = acc


@jax.jit
def kernel(inputs, table):
    batch, seq = inputs.shape
    grid = (batch // BLOCK_B,)
    return pl.pallas_call(
        _embed_block,
        grid=grid,
        in_specs=[
            pl.BlockSpec((BLOCK_B, seq), lambda i: (i, 0)),
            pl.BlockSpec((NUM_ROWS, EMB), lambda i: (0, 0)),
        ],
        out_specs=pl.BlockSpec((BLOCK_B, seq, EMB), lambda i: (i, 0, 0)),
        out_shape=jax.ShapeDtypeStruct((batch, seq, EMB), table.dtype),
        compiler_params=pltpu.CompilerParams(
            dimension_semantics=("parallel",),
        ),
    )(inputs, table)
